# baseline (speedup 1.0000x reference)
import jax
import jax.numpy as jnp
from jax.experimental import pallas as pl
from jax.experimental.pallas import tpu as pltpu


def _mlp3_kernel(x_ref, w1_ref, b1_ref, w2_ref, b2_ref, w3_ref, b3_ref, o_ref):
    x = x_ref[...].astype(jnp.bfloat16)
    h1 = jnp.dot(x, w1_ref[...], preferred_element_type=jnp.float32)
    h1 = jnp.maximum(h1 + b1_ref[...], 0.0).astype(jnp.bfloat16)
    h2 = jnp.dot(h1, w2_ref[...], preferred_element_type=jnp.float32)
    h2 = jnp.maximum(h2 + b2_ref[...], 0.0).astype(jnp.bfloat16)
    y = jnp.dot(h2, w3_ref[...], preferred_element_type=jnp.float32)
    o_ref[...] = y + b3_ref[...]


def _mlp3(x, w1b, b1r, w2b, b2r, w3p, b3p, *, tm):
    M, K = x.shape
    N = w1b.shape[1]
    OP = w3p.shape[1]
    flops = 2 * M * K * N + 2 * M * N * N + 2 * M * N * OP
    bytes_accessed = 4 * M * K + 2 * (K * N + N * N + N * OP) + 4 * M * OP

    return pl.pallas_call(
        _mlp3_kernel,
        out_shape=jax.ShapeDtypeStruct((M, OP), jnp.float32),
        grid=(M // tm,),
        in_specs=[
            pl.BlockSpec((tm, K), lambda i: (i, 0)),
            pl.BlockSpec((K, N), lambda i: (0, 0)),
            pl.BlockSpec((1, N), lambda i: (0, 0)),
            pl.BlockSpec((N, N), lambda i: (0, 0)),
            pl.BlockSpec((1, N), lambda i: (0, 0)),
            pl.BlockSpec((N, OP), lambda i: (0, 0)),
            pl.BlockSpec((1, OP), lambda i: (0, 0)),
        ],
        out_specs=pl.BlockSpec((tm, OP), lambda i: (i, 0)),
        compiler_params=pltpu.CompilerParams(
            dimension_semantics=("parallel",),
        ),
        cost_estimate=pl.CostEstimate(
            flops=flops, transcendentals=0, bytes_accessed=bytes_accessed
        ),
    )(x, w1b, b1r, w2b, b2r, w3p, b3p)


@jax.jit
def kernel(x, w1, b1, w2, b2, w3, b3):
    M, K = x.shape
    N = w1.shape[1]
    O = w3.shape[1]
    OP = 128

    w1b = w1.astype(jnp.bfloat16)
    w2b = w2.astype(jnp.bfloat16)
    w3p = jnp.pad(w3.astype(jnp.bfloat16), ((0, 0), (0, OP - O)))
    b3p = jnp.pad(b3, (0, OP - O)).reshape(1, OP)
    b1r = b1.reshape(1, N)
    b2r = b2.reshape(1, N)

    tm = min(256, M)
    yp = _mlp3(x, w1b, b1r, w2b, b2r, w3p, b3p, tm=tm)
    return yp[:, :O]


# speedup vs baseline: 1.2275x; 1.2275x over previous
import jax
import jax.numpy as jnp
from jax.experimental import pallas as pl
from jax.experimental.pallas import tpu as pltpu

_NC = 512


def _mlp3_kernel(x_ref, w1_ref, b1_ref, w2_ref, b2_ref, w3_ref, b3_ref,
                 o_ref, xb_ref, h1_ref, h2_ref):
    n = w1_ref.shape[1]
    xb_ref[...] = x_ref[...].astype(jnp.bfloat16)
    for j in range(n // _NC):
        sl = slice(j * _NC, (j + 1) * _NC)
        w1c = w1_ref[:, sl].astype(jnp.bfloat16)
        acc = jnp.dot(xb_ref[...], w1c, preferred_element_type=jnp.float32)
        h1_ref[:, sl] = jnp.maximum(acc + b1_ref[:, sl], 0.0).astype(jnp.bfloat16)
    for j in range(n // _NC):
        sl = slice(j * _NC, (j + 1) * _NC)
        w2c = w2_ref[:, sl].astype(jnp.bfloat16)
        acc = jnp.dot(h1_ref[...], w2c, preferred_element_type=jnp.float32)
        h2_ref[:, sl] = jnp.maximum(acc + b2_ref[:, sl], 0.0).astype(jnp.bfloat16)
    w3c = w3_ref[...].astype(jnp.bfloat16)
    y = jnp.dot(h2_ref[...], w3c, preferred_element_type=jnp.float32)
    o_ref[...] = y + b3_ref[...]


def _mlp3(x, w1, b1r, w2, b2r, w3, b3r, *, tm):
    M, K = x.shape
    N = w1.shape[1]
    O = w3.shape[1]
    flops = 2 * M * K * N + 2 * M * N * N + 2 * M * N * O
    bytes_accessed = 4 * (M * K + K * N + N * N + N * O + M * O)

    return pl.pallas_call(
        _mlp3_kernel,
        out_shape=jax.ShapeDtypeStruct((M, O), jnp.float32),
        grid=(M // tm,),
        in_specs=[
            pl.BlockSpec((tm, K), lambda i: (i, 0)),
            pl.BlockSpec((K, N), lambda i: (0, 0)),
            pl.BlockSpec((1, N), lambda i: (0, 0)),
            pl.BlockSpec((N, N), lambda i: (0, 0)),
            pl.BlockSpec((1, N), lambda i: (0, 0)),
            pl.BlockSpec((N, O), lambda i: (0, 0)),
            pl.BlockSpec((1, O), lambda i: (0, 0)),
        ],
        out_specs=pl.BlockSpec((tm, O), lambda i: (i, 0)),
        scratch_shapes=[
            pltpu.VMEM((tm, K), jnp.bfloat16),
            pltpu.VMEM((tm, N), jnp.bfloat16),
            pltpu.VMEM((tm, N), jnp.bfloat16),
        ],
        compiler_params=pltpu.CompilerParams(
            dimension_semantics=("parallel",),
        ),
        cost_estimate=pl.CostEstimate(
            flops=flops, transcendentals=0, bytes_accessed=bytes_accessed
        ),
    )(x, w1, b1r, w2, b2r, w3, b3r)


@jax.jit
def kernel(x, w1, b1, w2, b2, w3, b3):
    M = x.shape[0]
    N = w1.shape[1]
    O = w3.shape[1]
    tm = min(512, M)
    return _mlp3(x, w1, b1.reshape(1, N), w2, b2.reshape(1, N),
                 w3, b3.reshape(1, O), tm=tm)
